# Initial kernel scaffold; baseline (speedup 1.0000x reference)
import jax
import jax.numpy as jnp
from jax.experimental import pallas as pl


def kernel(data, global_embs, sense_embs, ctx_weight):
    raise NotImplementedError("write your pallas kernel here")


# speedup vs baseline: 1.3446x; 1.3446x over previous
import functools

import jax
import jax.numpy as jnp
from jax import lax
from jax.experimental import pallas as pl
from jax.experimental.pallas import tpu as pltpu
from jax.experimental.pallas import tpu_sc as plsc

B = 4096
D = 64
NCOL = 22
W2 = 10
NEG = 5
NC = 2
NS = 16
NW = NC * NS
PER_W = B // NW
CHUNK = 64
NCH = PER_W // CHUNK
L = 16
NROW = 2 * NEG + 1


def _sc_body(data_hbm, gl_hbm, se_hbm, w_hbm, out_hbm,
             data_v, ctx_idx_v, sense_idx_v, ctx_rows_v, sense_rows_v,
             w_v, out_v, sem):
    wid = lax.axis_index("s") * NC + lax.axis_index("c")
    pltpu.sync_copy(w_hbm, w_v)
    iota = lax.iota(jnp.int32, L)

    for c in range(NCH):
        base = wid * PER_W + c * CHUNK
        pltpu.sync_copy(data_hbm.at[pl.ds(base * NCOL, CHUNK * NCOL)], data_v)

        for g in range(CHUNK // L):
            rows = (iota + g * L) * NCOL
            off = c * CHUNK + g * L
            for j in range(W2):
                vals = plsc.load_gather(data_v, [rows + j])
                ctx_idx_v[j, pl.ds(g * L, L)] = vals * 2
            vals = plsc.load_gather(data_v, [rows + (W2 + 1)])
            sense_idx_v[0, pl.ds(g * L, L)] = vals * 2
            for n in range(NEG):
                vals = plsc.load_gather(data_v, [rows + (W2 + 2 + n)])
                sense_idx_v[1 + n, pl.ds(g * L, L)] = vals * 2
                mvals = plsc.load_gather(data_v, [rows + (W2 + 2 + NEG + n)])
                out_v[1 + NEG + n, pl.ds(off, L)] = mvals.astype(jnp.float32)

        copies = []
        for j in range(W2):
            copies.append(
                pltpu.async_copy(gl_hbm.at[ctx_idx_v.at[j]],
                                 ctx_rows_v.at[j], sem))
        for r in range(1 + NEG):
            copies.append(
                pltpu.async_copy(se_hbm.at[sense_idx_v.at[r]],
                                 sense_rows_v.at[r], sem))
        for cp in copies:
            cp.wait()

        for g in range(CHUNK // L):
            s_idx = iota + g * L

            def dbody(d, carry):
                accp = carry[0]
                accn = carry[1:]
                dvec = jnp.full((L,), d, jnp.int32)
                feat = jnp.zeros((L,), jnp.float32)
                for j in range(W2):
                    jvec = jnp.full((L,), j, jnp.int32)
                    v = plsc.load_gather(ctx_rows_v, [jvec, s_idx, dvec])
                    wv = plsc.load_gather(w_v, [jnp.full((L,), j * D, jnp.int32) + dvec])
                    feat = feat + v * wv
                pv = plsc.load_gather(
                    sense_rows_v, [jnp.full((L,), 0, jnp.int32), s_idx, dvec])
                new = [accp + feat * pv]
                for n in range(NEG):
                    nv = plsc.load_gather(
                        sense_rows_v,
                        [jnp.full((L,), 1 + n, jnp.int32), s_idx, dvec])
                    new.append(accn[n] + feat * nv)
                return tuple(new)

            z = jnp.zeros((L,), jnp.float32)
            accs = lax.fori_loop(0, D, dbody, (z,) * (1 + NEG))
            off = c * CHUNK + g * L
            for r in range(1 + NEG):
                out_v[r, pl.ds(off, L)] = accs[r]

    pltpu.sync_copy(out_v, out_hbm.at[wid])


def _sc_ips(data_flat, gl, se, w_flat):
    mesh = plsc.VectorSubcoreMesh(
        core_axis_name="c", subcore_axis_name="s",
        num_cores=NC, num_subcores=NS)
    f = pl.kernel(
        _sc_body,
        out_type=jax.ShapeDtypeStruct((NW, NROW, PER_W), jnp.float32),
        mesh=mesh,
        compiler_params=pltpu.CompilerParams(
            needs_layout_passes=False, use_tc_tiling_on_sc=False),
        scratch_types=[
            pltpu.VMEM((CHUNK * NCOL,), jnp.int32),
            pltpu.VMEM((W2, CHUNK), jnp.int32),
            pltpu.VMEM((1 + NEG, CHUNK), jnp.int32),
            pltpu.VMEM((W2, CHUNK, D), jnp.float32),
            pltpu.VMEM((1 + NEG, CHUNK, D), jnp.float32),
            pltpu.VMEM((W2 * D,), jnp.float32),
            pltpu.VMEM((NROW, PER_W), jnp.float32),
            pltpu.SemaphoreType.DMA,
        ],
    )
    return f(data_flat, gl, se, w_flat)


def _tc_loss_body(x_ref, pos_ref, neg_ref):
    x = x_ref[...]
    pos = x[:, 0:1, :]
    negs = x[:, 1:1 + NEG, :]
    mask = x[:, 1 + NEG:NROW, :]

    def softplus(t):
        return jnp.maximum(t, 0.0) + jnp.log1p(jnp.exp(-jnp.abs(t)))

    pos_terms = softplus(jnp.clip(-pos, -10.0, 10.0))
    neg_terms = mask * softplus(jnp.clip(negs, -10.0, 10.0))
    pos_ref[0, 0] = jnp.sum(pos_terms)
    neg_ref[0, 0] = jnp.sum(neg_terms)


def kernel(data, global_embs, sense_embs, ctx_weight):
    data_flat = data.astype(jnp.int32).reshape(-1)
    w_flat = ctx_weight.reshape(-1)
    ips3 = _sc_ips(data_flat, global_embs, sense_embs, w_flat)
    pos, neg = pl.pallas_call(
        _tc_loss_body,
        out_shape=(jax.ShapeDtypeStruct((1, 1), jnp.float32),
                   jax.ShapeDtypeStruct((1, 1), jnp.float32)),
        in_specs=[pl.BlockSpec(memory_space=pltpu.MemorySpace.VMEM)],
        out_specs=(pl.BlockSpec(memory_space=pltpu.MemorySpace.SMEM),
                   pl.BlockSpec(memory_space=pltpu.MemorySpace.SMEM)),
    )(ips3)
    return (pos[0, 0], neg[0, 0])
